# Initial kernel scaffold; baseline (speedup 1.0000x reference)
import jax
import jax.numpy as jnp
from jax.experimental import pallas as pl


def kernel(x, edge_index, Wx, bx, Wh, bh, w_c, b_g, W_lin, b_lin):
    raise NotImplementedError("write your pallas kernel here")


# speedup vs baseline: 17.7177x; 17.7177x over previous
import functools

import jax
import jax.numpy as jnp
from jax import lax
from jax.experimental import pallas as pl
from jax.experimental.pallas import tpu as pltpu
from jax.experimental.pallas import tpu_sc as plsc

N_NODES = 10000
E_EDGES = 320000
DIN = 128
DH = 64
DHALF = DIN // 2

NC, NS = 2, 16
CH = 128
NCHUNK = 160
EP_T = CH * NCHUNK
E_PAD = NS * EP_T
NIROWS = E_PAD // CH
ACC_ROWS = 10240
ZROWS = ACC_ROWS // NS


def _sc_mesh():
  return plsc.VectorSubcoreMesh(
      core_axis_name="c", subcore_axis_name="s", num_cores=NC, num_subcores=NS)


@functools.cache
def _sc_degree_kernel():
  return pl.kernel(
      _sc_degree_body,
      out_type=jax.ShapeDtypeStruct((NC, ACC_ROWS), jnp.float32),
      mesh=_sc_mesh(),
      scratch_types=[
          pltpu.VMEM((NIROWS // (NC * NS), CH), jnp.int32),
          pltpu.VMEM((CH,), jnp.float32),
          pltpu.VMEM_SHARED((ACC_ROWS,), jnp.float32),
      ],
  )


def _sc_degree_body(rowd_hbm, zeros1d_hbm, ones_hbm, out_hbm, idx_v, ones_v,
                    acc):
  c = lax.axis_index("c")
  s = lax.axis_index("s")
  wid = s * NC + c
  nck = NIROWS // (NC * NS)
  pltpu.sync_copy(ones_hbm, ones_v)
  pltpu.sync_copy(zeros1d_hbm, acc.at[pl.ds(s * ZROWS, ZROWS)])
  pltpu.sync_copy(rowd_hbm.at[pl.ds(wid * nck, nck)], idx_v)
  plsc.subcore_barrier()

  @pl.loop(0, nck)
  def _(j):
    pltpu.sync_copy(ones_v, acc.at[idx_v.at[j]], add=True)

  plsc.subcore_barrier()
  pltpu.sync_copy(acc.at[pl.ds(s * ZROWS, ZROWS)],
                  out_hbm.at[c, pl.ds(s * ZROWS, ZROWS)])


@functools.cache
def _sc_propagate_kernel():
  return pl.kernel(
      _sc_propagate_body,
      out_type=jax.ShapeDtypeStruct((NC, ACC_ROWS, DHALF), jnp.float32),
      mesh=_sc_mesh(),
      compiler_params=pltpu.CompilerParams(use_tc_tiling_on_sc=False),
      scratch_types=[
          pltpu.VMEM((NCHUNK, CH), jnp.int32),
          pltpu.VMEM((NCHUNK, CH), jnp.int32),
          pltpu.VMEM((CH, DHALF), jnp.float32),
          pltpu.VMEM((CH, DHALF), jnp.float32),
          pltpu.VMEM_SHARED((ACC_ROWS, DHALF), jnp.float32),
          pltpu.SemaphoreType.DMA,
          pltpu.SemaphoreType.DMA,
      ],
  )


def _sc_propagate_body(xsp_hbm, rowg_hbm, col_hbm, zeros2d_hbm, out_hbm,
                       rowi_v, coli_v, buf_a, buf_b, acc, sem_a, sem_b):
  c = lax.axis_index("c")
  s = lax.axis_index("s")
  pltpu.sync_copy(rowg_hbm.at[c, pl.ds(s * NCHUNK, NCHUNK)], rowi_v)
  pltpu.sync_copy(col_hbm.at[pl.ds(s * NCHUNK, NCHUNK)], coli_v)
  pltpu.sync_copy(zeros2d_hbm, acc.at[pl.ds(s * ZROWS, ZROWS)])
  plsc.subcore_barrier()

  pltpu.async_copy(xsp_hbm.at[rowi_v.at[0]], buf_a, sem_a)
  pltpu.async_copy(xsp_hbm.at[rowi_v.at[1]], buf_b, sem_b)

  @pl.loop(0, NCHUNK, step=2)
  def _(j):
    pltpu.make_async_copy(xsp_hbm.at[rowi_v.at[j]], buf_a, sem_a).wait()
    pltpu.sync_copy(buf_a, acc.at[coli_v.at[j]], add=True)

    @pl.when(j + 2 < NCHUNK)
    def _():
      pltpu.async_copy(xsp_hbm.at[rowi_v.at[j + 2]], buf_a, sem_a)

    pltpu.make_async_copy(xsp_hbm.at[rowi_v.at[j + 1]], buf_b, sem_b).wait()
    pltpu.sync_copy(buf_b, acc.at[coli_v.at[j + 1]], add=True)

    @pl.when(j + 3 < NCHUNK)
    def _():
      pltpu.async_copy(xsp_hbm.at[rowi_v.at[j + 3]], buf_b, sem_b)

  plsc.subcore_barrier()
  pltpu.sync_copy(acc.at[pl.ds(s * ZROWS, ZROWS)],
                  out_hbm.at[c, pl.ds(s * ZROWS, ZROWS)])


def _tc_scale_body(x_ref, degp_ref, xsp_ref, dinv_ref):
  deg = degp_ref[0] + degp_ref[1]
  dinv = jnp.where(deg > 0.0, lax.rsqrt(jnp.maximum(deg, 1e-12)), 0.0)
  dinv_ref[...] = dinv
  xs = x_ref[...] * dinv
  xsp_ref[0] = xs[:, :DHALF]
  xsp_ref[1] = xs[:, DHALF:]


def _tc_scale(x, degp3):
  R = 400
  return pl.pallas_call(
      _tc_scale_body,
      grid=(N_NODES // R,),
      in_specs=[
          pl.BlockSpec((R, DIN), lambda i: (i, 0)),
          pl.BlockSpec((NC, R, 1), lambda i: (0, i, 0)),
      ],
      out_specs=[
          pl.BlockSpec((NC, R, DHALF), lambda i: (0, i, 0)),
          pl.BlockSpec((R, 1), lambda i: (i, 0)),
      ],
      out_shape=[
          jax.ShapeDtypeStruct((NC, N_NODES, DHALF), jnp.float32),
          jax.ShapeDtypeStruct((N_NODES, 1), jnp.float32),
      ],
  )(x, degp3)


def _tc_gates_body(x_ref, s_ref, dinv_ref, w0_ref, w1t_ref, w1b_ref, bias_ref,
                   wc2_ref, wl_ref, blin_ref, out_ref):
  x = x_ref[...]
  dinv = dinv_ref[...]
  px0 = -dinv * s_ref[0]
  px1 = -dinv * s_ref[1]
  dot = lambda a, b: jnp.dot(a, b, preferred_element_type=jnp.float32)

  def gate(g):
    return (dot(x, w0_ref[g]) + dot(px0, w1t_ref[g]) + dot(px1, w1b_ref[g])
            + bias_ref[0, g])

  gi, gc, go = gate(0), gate(1), gate(2)
  cs = jax.nn.sigmoid(gi) * jnp.tanh(gc)
  o = jax.nn.sigmoid(go + wc2_ref[...] * cs)
  h = o * jnp.tanh(cs)
  out_ref[...] = jax.nn.sigmoid(
      jnp.sum(h * wl_ref[...], axis=1, keepdims=True) + blin_ref[...])


def _tc_gates(x, s_part, dinv, w0, w1t, w1b, bias, wc2, wl, blin):
  R = 400
  full = lambda shape: pl.BlockSpec(shape, lambda i: tuple(0 for _ in shape))
  return pl.pallas_call(
      _tc_gates_body,
      grid=(N_NODES // R,),
      in_specs=[
          pl.BlockSpec((R, DIN), lambda i: (i, 0)),
          pl.BlockSpec((NC, R, DHALF), lambda i: (0, i, 0)),
          pl.BlockSpec((R, 1), lambda i: (i, 0)),
          full((3, DIN, DH)), full((3, DHALF, DH)), full((3, DHALF, DH)),
          full((1, 3, DH)), full((1, DH)), full((1, DH)), full((1, 1)),
      ],
      out_specs=pl.BlockSpec((R, 1), lambda i: (i, 0)),
      out_shape=jax.ShapeDtypeStruct((N_NODES, 1), jnp.float32),
  )(x, s_part, dinv, w0, w1t, w1b, bias, wc2, wl, blin)


def kernel(x, edge_index, Wx, bx, Wh, bh, w_c, b_g, W_lin, b_lin):
  row = edge_index[0]
  col = edge_index[1]
  npad = E_PAD - E_EDGES
  pad0 = jnp.zeros((npad,), jnp.int32)
  padd = jnp.full((npad,), N_NODES, jnp.int32)
  rowg = jnp.concatenate([row, pad0]).reshape(NIROWS, CH)
  rowg2 = jnp.stack([rowg, rowg + N_NODES])
  rowd = jnp.concatenate([row, padd]).reshape(NIROWS, CH)
  cols = jnp.concatenate([col, padd]).reshape(NIROWS, CH)
  zeros1d = jnp.zeros((ZROWS,), jnp.float32)
  zeros2d = jnp.zeros((ZROWS, DHALF), jnp.float32)
  ones1d = jnp.ones((CH,), jnp.float32)

  degp = _sc_degree_kernel()(rowd, zeros1d, ones1d)
  degp3 = degp[:, :N_NODES, None]
  xsp, dinv = _tc_scale(x, degp3)
  xs_flat = xsp.reshape(NC * N_NODES, DHALF)
  s_part = _sc_propagate_kernel()(xs_flat, rowg2, cols, zeros2d)[:, :N_NODES]

  gsel = jnp.array([0, 2, 3])
  w0 = Wx[gsel, 0]
  w1t = Wx[gsel, 1, :DHALF]
  w1b = Wx[gsel, 1, DHALF:]
  bias = (bx + bh + b_g)[gsel][None]
  out = _tc_gates(x, s_part, dinv, w0, w1t, w1b, bias,
                  w_c[2][None, :], W_lin.reshape(1, DH), b_lin.reshape(1, 1))
  return out


# speedup vs baseline: 18.1379x; 1.0237x over previous
import functools

import jax
import jax.numpy as jnp
from jax import lax
from jax.experimental import pallas as pl
from jax.experimental.pallas import tpu as pltpu
from jax.experimental.pallas import tpu_sc as plsc

N_NODES = 10000
E_EDGES = 320000
DIN = 128
DH = 64
DHALF = DIN // 2

NC, NS = 2, 16
CH = 128
NCHUNK = 160
EP_T = CH * NCHUNK
E_PAD = NS * EP_T
NIROWS = E_PAD // CH
ACC_ROWS = 10240
ZROWS = ACC_ROWS // NS
NBUF = 4
PREF = 2


def _sc_mesh():
  return plsc.VectorSubcoreMesh(
      core_axis_name="c", subcore_axis_name="s", num_cores=NC, num_subcores=NS)


@functools.cache
def _sc_degree_kernel():
  return pl.kernel(
      _sc_degree_body,
      out_type=jax.ShapeDtypeStruct((NC, ACC_ROWS), jnp.float32),
      mesh=_sc_mesh(),
      scratch_types=[
          pltpu.VMEM((NIROWS // (NC * NS), CH), jnp.int32),
          pltpu.VMEM((CH,), jnp.float32),
          pltpu.VMEM_SHARED((ACC_ROWS,), jnp.float32),
      ],
  )


def _sc_degree_body(rowd_hbm, zeros1d_hbm, ones_hbm, out_hbm, idx_v, ones_v,
                    acc):
  c = lax.axis_index("c")
  s = lax.axis_index("s")
  wid = s * NC + c
  nck = NIROWS // (NC * NS)
  pltpu.sync_copy(ones_hbm, ones_v)
  pltpu.sync_copy(zeros1d_hbm, acc.at[pl.ds(s * ZROWS, ZROWS)])
  pltpu.sync_copy(rowd_hbm.at[pl.ds(wid * nck, nck)], idx_v)
  plsc.subcore_barrier()

  @pl.loop(0, nck)
  def _(j):
    pltpu.sync_copy(ones_v, acc.at[idx_v.at[j]], add=True)

  plsc.subcore_barrier()
  pltpu.sync_copy(acc.at[pl.ds(s * ZROWS, ZROWS)],
                  out_hbm.at[c, pl.ds(s * ZROWS, ZROWS)])


@functools.cache
def _sc_propagate_kernel():
  return pl.kernel(
      _sc_propagate_body,
      out_type=jax.ShapeDtypeStruct((NC, ACC_ROWS, DHALF), jnp.float32),
      mesh=_sc_mesh(),
      compiler_params=pltpu.CompilerParams(use_tc_tiling_on_sc=False),
      scratch_types=[
          pltpu.VMEM((NCHUNK, CH), jnp.int32),
          pltpu.VMEM((NCHUNK, CH), jnp.int32),
          [pltpu.VMEM((CH, DHALF), jnp.float32) for _ in range(NBUF)],
          [pltpu.SemaphoreType.DMA for _ in range(NBUF)],
          pltpu.VMEM_SHARED((ACC_ROWS, DHALF), jnp.float32),
      ],
  )


def _sc_propagate_body(xsp_hbm, rowg_hbm, col_hbm, zeros2d_hbm, out_hbm,
                       rowi_v, coli_v, bufs, gsem, acc):
  c = lax.axis_index("c")
  s = lax.axis_index("s")
  pltpu.sync_copy(rowg_hbm.at[c, pl.ds(s * NCHUNK, NCHUNK)], rowi_v)
  pltpu.sync_copy(col_hbm.at[pl.ds(s * NCHUNK, NCHUNK)], coli_v)
  pltpu.sync_copy(zeros2d_hbm, acc.at[pl.ds(s * ZROWS, ZROWS)])
  plsc.subcore_barrier()

  for b in range(NBUF):
    pltpu.async_copy(xsp_hbm.at[rowi_v.at[b]], bufs[b], gsem[b])

  @pl.loop(0, NCHUNK, step=NBUF)
  def _(j):
    for b in range(NBUF):
      k = j + b
      pltpu.make_async_copy(xsp_hbm.at[rowi_v.at[k]], bufs[b], gsem[b]).wait()
      pltpu.sync_copy(bufs[b], acc.at[coli_v.at[k]], add=True)

      @pl.when(k + NBUF < NCHUNK)
      def _():
        pltpu.async_copy(xsp_hbm.at[rowi_v.at[k + NBUF]], bufs[b], gsem[b])

  plsc.subcore_barrier()
  pltpu.sync_copy(acc.at[pl.ds(s * ZROWS, ZROWS)],
                  out_hbm.at[c, pl.ds(s * ZROWS, ZROWS)])


def _tc_scale_body(x_ref, degp_ref, xsp_ref, dinv_ref):
  deg = degp_ref[0] + degp_ref[1]
  dinv = jnp.where(deg > 0.0, lax.rsqrt(jnp.maximum(deg, 1e-12)), 0.0)
  dinv_ref[...] = dinv
  xs = x_ref[...] * dinv
  xsp_ref[0] = xs[:, :DHALF]
  xsp_ref[1] = xs[:, DHALF:]


def _tc_scale(x, degp3):
  R = 400
  return pl.pallas_call(
      _tc_scale_body,
      grid=(N_NODES // R,),
      in_specs=[
          pl.BlockSpec((R, DIN), lambda i: (i, 0)),
          pl.BlockSpec((NC, R, 1), lambda i: (0, i, 0)),
      ],
      out_specs=[
          pl.BlockSpec((NC, R, DHALF), lambda i: (0, i, 0)),
          pl.BlockSpec((R, 1), lambda i: (i, 0)),
      ],
      out_shape=[
          jax.ShapeDtypeStruct((NC, N_NODES, DHALF), jnp.float32),
          jax.ShapeDtypeStruct((N_NODES, 1), jnp.float32),
      ],
  )(x, degp3)


def _tc_gates_body(x_ref, s_ref, dinv_ref, w0_ref, w1t_ref, w1b_ref, bias_ref,
                   wc2_ref, wl_ref, blin_ref, out_ref):
  x = x_ref[...]
  dinv = dinv_ref[...]
  px0 = -dinv * s_ref[0]
  px1 = -dinv * s_ref[1]
  dot = lambda a, b: jnp.dot(a, b, preferred_element_type=jnp.float32)

  def gate(g):
    return (dot(x, w0_ref[g]) + dot(px0, w1t_ref[g]) + dot(px1, w1b_ref[g])
            + bias_ref[0, g])

  gi, gc, go = gate(0), gate(1), gate(2)
  cs = jax.nn.sigmoid(gi) * jnp.tanh(gc)
  o = jax.nn.sigmoid(go + wc2_ref[...] * cs)
  h = o * jnp.tanh(cs)
  out_ref[...] = jax.nn.sigmoid(
      jnp.sum(h * wl_ref[...], axis=1, keepdims=True) + blin_ref[...])


def _tc_gates(x, s_part, dinv, w0, w1t, w1b, bias, wc2, wl, blin):
  R = 400
  full = lambda shape: pl.BlockSpec(shape, lambda i: tuple(0 for _ in shape))
  return pl.pallas_call(
      _tc_gates_body,
      grid=(N_NODES // R,),
      in_specs=[
          pl.BlockSpec((R, DIN), lambda i: (i, 0)),
          pl.BlockSpec((NC, R, DHALF), lambda i: (0, i, 0)),
          pl.BlockSpec((R, 1), lambda i: (i, 0)),
          full((3, DIN, DH)), full((3, DHALF, DH)), full((3, DHALF, DH)),
          full((1, 3, DH)), full((1, DH)), full((1, DH)), full((1, 1)),
      ],
      out_specs=pl.BlockSpec((R, 1), lambda i: (i, 0)),
      out_shape=jax.ShapeDtypeStruct((N_NODES, 1), jnp.float32),
  )(x, s_part, dinv, w0, w1t, w1b, bias, wc2, wl, blin)


def kernel(x, edge_index, Wx, bx, Wh, bh, w_c, b_g, W_lin, b_lin):
  row = edge_index[0]
  col = edge_index[1]
  npad = E_PAD - E_EDGES
  pad0 = jnp.zeros((npad,), jnp.int32)
  padd = jnp.full((npad,), N_NODES, jnp.int32)
  rowg = jnp.concatenate([row, pad0]).reshape(NIROWS, CH)
  rowg2 = jnp.stack([rowg, rowg + N_NODES])
  rowd = jnp.concatenate([row, padd]).reshape(NIROWS, CH)
  cols = jnp.concatenate([col, padd]).reshape(NIROWS, CH)
  zeros1d = jnp.zeros((ZROWS,), jnp.float32)
  zeros2d = jnp.zeros((ZROWS, DHALF), jnp.float32)
  ones1d = jnp.ones((CH,), jnp.float32)

  degp = _sc_degree_kernel()(rowd, zeros1d, ones1d)
  degp3 = degp[:, :N_NODES, None]
  xsp, dinv = _tc_scale(x, degp3)
  xs_flat = xsp.reshape(NC * N_NODES, DHALF)
  s_part = _sc_propagate_kernel()(xs_flat, rowg2, cols, zeros2d)[:, :N_NODES]

  gsel = jnp.array([0, 2, 3])
  w0 = Wx[gsel, 0]
  w1t = Wx[gsel, 1, :DHALF]
  w1b = Wx[gsel, 1, DHALF:]
  bias = (bx + bh + b_g)[gsel][None]
  out = _tc_gates(x, s_part, dinv, w0, w1t, w1b, bias,
                  w_c[2][None, :], W_lin.reshape(1, DH), b_lin.reshape(1, 1))
  return out
